# baseline (speedup 1.0000x reference)
import functools

import jax
import jax.numpy as jnp
from jax import lax
from jax.experimental import pallas as pl
from jax.experimental.pallas import tpu as pltpu
from jax.experimental.pallas import tpu_sc as plsc

D_MODEL = 1024
HIDDEN = 2048
N_EXPERTS = 8
TOP_K = 2
N_TOK = 2048
A = N_TOK * TOP_K
BT = 256
G_MAX = A // BT + N_EXPERTS
P = G_MAX * BT

NC = 2
NS = 16
NW = NC * NS


def _sc_mesh():
    return plsc.VectorSubcoreMesh(core_axis_name="c", subcore_axis_name="s",
                                  num_cores=NC, num_subcores=NS)


_NCHUNK = A // 128


def _router_body(x_ref, wr_ref, dest_ref, w_ref, binfo_ref):
    x = x_ref[...]
    wr = wr_ref[...]
    logits = lax.dot_general(x, wr, (((1,), (1,)), ((), ())),
                             preferred_element_type=jnp.float32)
    iota = lax.broadcasted_iota(jnp.int32, logits.shape, 1)
    m1 = jnp.max(logits, axis=1, keepdims=True)
    i1 = jnp.min(jnp.where(logits == m1, iota, N_EXPERTS), axis=1, keepdims=True)
    masked = jnp.where(iota == i1, -jnp.inf, logits)
    m2 = jnp.max(masked, axis=1, keepdims=True)
    i2 = jnp.min(jnp.where(masked == m2, iota, N_EXPERTS), axis=1, keepdims=True)
    e2 = jnp.exp(m2 - m1)
    denom = 1.0 + e2
    w_ref[...] = jnp.concatenate([1.0 / denom, e2 / denom], axis=1)

    e_col = jnp.concatenate([i1, i2], axis=0)
    e3 = e_col.reshape(_NCHUNK, 128, 1)
    iota_e = lax.broadcasted_iota(jnp.int32, (_NCHUNK, 128, N_EXPERTS), 2)
    oh = (e3 == iota_e).astype(jnp.float32)
    ii = lax.broadcasted_iota(jnp.int32, (128, 128), 0)
    jj = lax.broadcasted_iota(jnp.int32, (128, 128), 1)
    tril = (jj < ii).astype(jnp.float32)
    tril_b = jnp.broadcast_to(tril[None], (_NCHUNK, 128, 128))
    excl = lax.dot_general(tril_b, oh, (((2,), (1,)), ((0,), (0,))),
                           preferred_element_type=jnp.float32)
    chunk_sums = excl[:, 127, :] + oh[:, 127, :]
    ii32 = lax.broadcasted_iota(jnp.int32, (_NCHUNK, _NCHUNK), 0)
    jj32 = lax.broadcasted_iota(jnp.int32, (_NCHUNK, _NCHUNK), 1)
    tril32 = (jj32 < ii32).astype(jnp.float32)
    offs = lax.dot_general(tril32, chunk_sums, (((1,), (0,)), ((), ())),
                           preferred_element_type=jnp.float32)
    counts = jnp.sum(chunk_sums, axis=0, keepdims=True)
    nb = (counts.astype(jnp.int32) + (BT - 1)) >> 8
    nb_f = nb.astype(jnp.float32)
    ii8 = lax.broadcasted_iota(jnp.int32, (N_EXPERTS, N_EXPERTS), 0)
    jj8 = lax.broadcasted_iota(jnp.int32, (N_EXPERTS, N_EXPERTS), 1)
    sup8 = (ii8 < jj8).astype(jnp.float32)
    bexcl = lax.dot_general(nb_f, sup8, (((1,), (0,)), ((), ())),
                            preferred_element_type=jnp.float32)
    base = float(BT) * bexcl
    dest3 = excl + offs[:, None, :] + base[None, :, :]
    dest_ref[...] = jnp.sum(dest3 * oh, axis=2).astype(jnp.int32)
    binfo_ref[...] = jnp.concatenate([bexcl, bexcl + nb_f],
                                     axis=0).astype(jnp.int32)


def _router(x_flat, Wr):
    return pl.pallas_call(
        _router_body,
        out_shape=(
            jax.ShapeDtypeStruct((_NCHUNK, 128), jnp.int32),
            jax.ShapeDtypeStruct((N_TOK, TOP_K), jnp.float32),
            jax.ShapeDtypeStruct((2, N_EXPERTS), jnp.int32),
        ),
    )(x_flat, Wr)


_AS_W = A // NW
_CH_D = 16
_NCH_D = _AS_W // _CH_D


@functools.cache
def _make_sc_dispatch():
    @functools.partial(
        pl.kernel,
        out_type=jax.ShapeDtypeStruct((P, D_MODEL), jnp.float32),
        mesh=_sc_mesh(),
        scratch_types=[
            pltpu.VMEM((_NCH_D, _CH_D), jnp.int32),
            pltpu.VMEM((_CH_D, D_MODEL), jnp.float32),
            pltpu.VMEM((_CH_D, D_MODEL), jnp.float32),
            pltpu.SemaphoreType.DMA,
            pltpu.SemaphoreType.DMA,
            pltpu.SemaphoreType.DMA,
            pltpu.SemaphoreType.DMA,
        ],
    )
    def _sc_dispatch(x_hbm, dest_hbm, out_hbm, dest_v, bufa, bufb,
                     gsa, gsb, ssa, ssb):
        wid = lax.axis_index("s") * NC + lax.axis_index("c")
        a_base = wid * _AS_W
        pltpu.sync_copy(dest_hbm.at[wid], dest_v)
        bufs = (bufa, bufb)
        gsems = (gsa, gsb)
        ssems = (ssa, ssb)

        def tok_idx(c):
            a_vec = (a_base + c * _CH_D) + lax.iota(jnp.int32, 16)
            return lax.bitwise_and(a_vec, N_TOK - 1)

        g_h = {}
        s_h = {}
        g_h[0] = pltpu.async_copy(x_hbm.at[tok_idx(0)], bufs[0], gsems[0])
        for c in range(_NCH_D):
            b = c % 2
            g_h[c].wait()
            if c + 1 < _NCH_D:
                if c >= 1:
                    s_h[c - 1].wait()
                g_h[c + 1] = pltpu.async_copy(
                    x_hbm.at[tok_idx(c + 1)], bufs[1 - b], gsems[1 - b])
            s_h[c] = pltpu.async_copy(bufs[b], out_hbm.at[dest_v.at[c]],
                                      ssems[b])
        s_h[_NCH_D - 2].wait()
        s_h[_NCH_D - 1].wait()

    return _sc_dispatch


_HS = HIDDEN // 2


def _mm_body(meta_ref, xs_ref, w1_ref, w3_ref, w2_ref, y_ref):
    g = pl.program_id(0)
    s = pl.program_id(1)
    valid = g < meta_ref[G_MAX]

    @pl.when(valid)
    def _():
        xb = xs_ref[...]
        a = lax.dot_general(xb, w1_ref[0], (((1,), (1,)), ((), ())),
                            preferred_element_type=jnp.float32)
        b = lax.dot_general(xb, w3_ref[0], (((1,), (1,)), ((), ())),
                            preferred_element_type=jnp.float32)
        h = a * jax.nn.sigmoid(a) * b
        y = lax.dot_general(h, w2_ref[0], (((1,), (1,)), ((), ())),
                            preferred_element_type=jnp.float32)

        @pl.when(s == 0)
        def _():
            y_ref[...] = y

        @pl.when(s != 0)
        def _():
            y_ref[...] += y


def _grouped_mm(meta, xs, W1, W3, W2):
    def xs_map(g, s, m):
        return (jnp.minimum(g, m[G_MAX] - 1), 0)

    def y_map(g, s, m):
        return (jnp.minimum(g, m[G_MAX] - 1), 0)

    grid_spec = pltpu.PrefetchScalarGridSpec(
        num_scalar_prefetch=1,
        grid=(G_MAX, 2),
        in_specs=[
            pl.BlockSpec((BT, D_MODEL), xs_map),
            pl.BlockSpec((1, _HS, D_MODEL), lambda g, s, m: (m[g], s, 0)),
            pl.BlockSpec((1, _HS, D_MODEL), lambda g, s, m: (m[g], s, 0)),
            pl.BlockSpec((1, D_MODEL, _HS), lambda g, s, m: (m[g], 0, s)),
        ],
        out_specs=pl.BlockSpec((BT, D_MODEL), y_map),
    )
    return pl.pallas_call(
        _mm_body,
        grid_spec=grid_spec,
        out_shape=jax.ShapeDtypeStruct((P, D_MODEL), jnp.float32),
        compiler_params=pltpu.CompilerParams(
            dimension_semantics=("arbitrary", "arbitrary")),
    )(meta, xs, W1, W3, W2)


_TOK_W = N_TOK // NW
_CH_C = 16
_NCH_C = _TOK_W // _CH_C
_NV = D_MODEL // 16
_UNROLL = 8


@functools.cache
def _make_sc_combine():
    @functools.partial(
        pl.kernel,
        out_type=jax.ShapeDtypeStruct((N_TOK, D_MODEL), jnp.float32),
        mesh=_sc_mesh(),
        scratch_types=[
            pltpu.VMEM((_TOK_W,), jnp.int32),
            pltpu.VMEM((_TOK_W,), jnp.int32),
            pltpu.VMEM((_TOK_W,), jnp.float32),
            pltpu.VMEM((_TOK_W,), jnp.float32),
            pltpu.VMEM((_CH_C, D_MODEL), jnp.float32),
            pltpu.VMEM((_CH_C, D_MODEL), jnp.float32),
            pltpu.VMEM((_CH_C, D_MODEL), jnp.float32),
            pltpu.VMEM((_CH_C, D_MODEL), jnp.float32),
            pltpu.SemaphoreType.DMA,
            pltpu.SemaphoreType.DMA,
            pltpu.SemaphoreType.DMA,
            pltpu.SemaphoreType.DMA,
            pltpu.SemaphoreType.DMA,
            pltpu.SemaphoreType.DMA,
        ],
    )
    def _sc_combine(y_hbm, pos0_hbm, pos1_hbm, w0_hbm, w1_hbm, out_hbm,
                    idx0_v, idx1_v, w0_v, w1_v, a0, a1, b0, b1,
                    gs0a, gs1a, gs0b, gs1b, wsa, wsb):
        wid = lax.axis_index("s") * NC + lax.axis_index("c")
        base = wid * _TOK_W
        pltpu.sync_copy(pos0_hbm.at[pl.ds(base, _TOK_W)], idx0_v)
        pltpu.sync_copy(pos1_hbm.at[pl.ds(base, _TOK_W)], idx1_v)
        pltpu.sync_copy(w0_hbm.at[pl.ds(base, _TOK_W)], w0_v)
        pltpu.sync_copy(w1_hbm.at[pl.ds(base, _TOK_W)], w1_v)
        e0 = (a0, b0)
        e1 = (a1, b1)
        gs0 = (gs0a, gs0b)
        gs1 = (gs1a, gs1b)
        ws = (wsa, wsb)

        def start_gathers(c, b):
            sl = pl.ds(c * _CH_C, _CH_C)
            return (pltpu.async_copy(y_hbm.at[idx0_v.at[sl]], e0[b], gs0[b]),
                    pltpu.async_copy(y_hbm.at[idx1_v.at[sl]], e1[b], gs1[b]))

        g_h = {0: start_gathers(0, 0)}
        w_h = {}
        for c in range(_NCH_C):
            b = c % 2
            g_h[c][0].wait()
            g_h[c][1].wait()
            if c + 1 < _NCH_C:
                if c >= 1:
                    w_h[c - 1].wait()
                g_h[c + 1] = start_gathers(c + 1, 1 - b)
            wvec0 = w0_v[pl.ds(c * _CH_C, _CH_C)]
            wvec1 = w1_v[pl.ds(c * _CH_C, _CH_C)]
            for t in range(_CH_C):
                w0b = wvec0[t]
                w1b = wvec1[t]

                def body(j, _, t=t, w0b=w0b, w1b=w1b, b=b):
                    for u in range(_UNROLL):
                        vsl = pl.ds(j * (16 * _UNROLL) + u * 16, 16)
                        e0[b][t, vsl] = (e0[b][t, vsl] * w0b
                                         + e1[b][t, vsl] * w1b)
                    return 0
                lax.fori_loop(0, _NV // _UNROLL, body, 0)
            w_h[c] = pltpu.async_copy(
                e0[b], out_hbm.at[pl.ds(base + c * _CH_C, _CH_C)], ws[b])
        w_h[_NCH_C - 2].wait()
        w_h[_NCH_C - 1].wait()

    return _sc_combine


def kernel(x, Wr, W1, W3, W2):
    bsz, seq_len, d_model = x.shape
    x_flat = x.reshape(-1, d_model)

    dest32, top_w, binfo = _router(x_flat, Wr)
    bexcl = binfo[0]
    eob = jnp.sum((jnp.arange(G_MAX)[:, None] >= bexcl[None, :])
                  .astype(jnp.int32), axis=1) - 1
    meta = jnp.concatenate([eob, binfo[1, N_EXPERTS - 1:]]).astype(jnp.int32)

    dest3 = dest32.reshape(NW, _NCH_D, _CH_D)
    dflat = dest32.reshape(A)
    pos0 = dflat[:N_TOK]
    pos1 = dflat[N_TOK:]

    xs = _make_sc_dispatch()(x_flat, dest3)
    ys = _grouped_mm(meta, xs, W1, W3, W2)
    out = _make_sc_combine()(ys, pos0, pos1,
                             top_w[:, 0], top_w[:, 1])
    return out.reshape(bsz, seq_len, d_model)


# speedup vs baseline: 1.3833x; 1.3833x over previous
import functools

import jax
import jax.numpy as jnp
from jax import lax
from jax.experimental import pallas as pl
from jax.experimental.pallas import tpu as pltpu
from jax.experimental.pallas import tpu_sc as plsc

D_MODEL = 1024
HIDDEN = 2048
N_EXPERTS = 8
TOP_K = 2
N_TOK = 2048
A = N_TOK * TOP_K
BT = 256
G_MAX = A // BT + N_EXPERTS
P = G_MAX * BT

NC = 2
NS = 16
NW = NC * NS


def _sc_mesh():
    return plsc.VectorSubcoreMesh(core_axis_name="c", subcore_axis_name="s",
                                  num_cores=NC, num_subcores=NS)


_NCHUNK = A // 128


def _router_body(x_ref, wr_ref, dest_ref, w_ref, binfo_ref):
    x = x_ref[...]
    wr = wr_ref[...]
    logits = lax.dot_general(x, wr, (((1,), (1,)), ((), ())),
                             preferred_element_type=jnp.float32)
    iota = lax.broadcasted_iota(jnp.int32, logits.shape, 1)
    m1 = jnp.max(logits, axis=1, keepdims=True)
    i1 = jnp.min(jnp.where(logits == m1, iota, N_EXPERTS), axis=1, keepdims=True)
    masked = jnp.where(iota == i1, -jnp.inf, logits)
    m2 = jnp.max(masked, axis=1, keepdims=True)
    i2 = jnp.min(jnp.where(masked == m2, iota, N_EXPERTS), axis=1, keepdims=True)
    e2 = jnp.exp(m2 - m1)
    denom = 1.0 + e2
    w_ref[...] = jnp.concatenate([1.0 / denom, e2 / denom], axis=1)

    e_col = jnp.concatenate([i1, i2], axis=0)
    e3 = e_col.reshape(_NCHUNK, 128, 1)
    iota_e = lax.broadcasted_iota(jnp.int32, (_NCHUNK, 128, N_EXPERTS), 2)
    oh = (e3 == iota_e).astype(jnp.float32)
    ii = lax.broadcasted_iota(jnp.int32, (128, 128), 0)
    jj = lax.broadcasted_iota(jnp.int32, (128, 128), 1)
    tril = (jj < ii).astype(jnp.float32)
    tril_b = jnp.broadcast_to(tril[None], (_NCHUNK, 128, 128))
    excl = lax.dot_general(tril_b, oh, (((2,), (1,)), ((0,), (0,))),
                           preferred_element_type=jnp.float32)
    chunk_sums = excl[:, 127, :] + oh[:, 127, :]
    ii32 = lax.broadcasted_iota(jnp.int32, (_NCHUNK, _NCHUNK), 0)
    jj32 = lax.broadcasted_iota(jnp.int32, (_NCHUNK, _NCHUNK), 1)
    tril32 = (jj32 < ii32).astype(jnp.float32)
    offs = lax.dot_general(tril32, chunk_sums, (((1,), (0,)), ((), ())),
                           preferred_element_type=jnp.float32)
    counts = jnp.sum(chunk_sums, axis=0, keepdims=True)
    nb = (counts.astype(jnp.int32) + (BT - 1)) >> 8
    nb_f = nb.astype(jnp.float32)
    ii8 = lax.broadcasted_iota(jnp.int32, (N_EXPERTS, N_EXPERTS), 0)
    jj8 = lax.broadcasted_iota(jnp.int32, (N_EXPERTS, N_EXPERTS), 1)
    sup8 = (ii8 < jj8).astype(jnp.float32)
    bexcl = lax.dot_general(nb_f, sup8, (((1,), (0,)), ((), ())),
                            preferred_element_type=jnp.float32)
    base = float(BT) * bexcl
    dest3 = excl + offs[:, None, :] + base[None, :, :]
    dest_ref[...] = jnp.sum(dest3 * oh, axis=2).astype(jnp.int32)
    binfo_ref[...] = jnp.concatenate([bexcl, bexcl + nb_f],
                                     axis=0).astype(jnp.int32)


def _router(x_flat, Wr):
    return pl.pallas_call(
        _router_body,
        out_shape=(
            jax.ShapeDtypeStruct((_NCHUNK, 128), jnp.int32),
            jax.ShapeDtypeStruct((N_TOK, TOP_K), jnp.float32),
            jax.ShapeDtypeStruct((2, N_EXPERTS), jnp.int32),
        ),
    )(x_flat, Wr)


_AS_W = A // NW
_CH_D = 16
_NCH_D = _AS_W // _CH_D


@functools.cache
def _make_sc_dispatch():
    @functools.partial(
        pl.kernel,
        out_type=jax.ShapeDtypeStruct((P, D_MODEL), jnp.float32),
        mesh=_sc_mesh(),
        scratch_types=[
            pltpu.VMEM((_NCH_D, _CH_D), jnp.int32),
            pltpu.VMEM((_CH_D, D_MODEL), jnp.float32),
            pltpu.VMEM((_CH_D, D_MODEL), jnp.float32),
            pltpu.SemaphoreType.DMA,
            pltpu.SemaphoreType.DMA,
            pltpu.SemaphoreType.DMA,
            pltpu.SemaphoreType.DMA,
        ],
    )
    def _sc_dispatch(x_hbm, dest_hbm, out_hbm, dest_v, bufa, bufb,
                     gsa, gsb, ssa, ssb):
        wid = lax.axis_index("s") * NC + lax.axis_index("c")
        a_base = wid * _AS_W
        pltpu.sync_copy(dest_hbm.at[wid], dest_v)
        bufs = (bufa, bufb)
        gsems = (gsa, gsb)
        ssems = (ssa, ssb)

        def tok_idx(c):
            a_vec = (a_base + c * _CH_D) + lax.iota(jnp.int32, 16)
            return lax.bitwise_and(a_vec, N_TOK - 1)

        g_h = {}
        s_h = {}
        g_h[0] = pltpu.async_copy(x_hbm.at[tok_idx(0)], bufs[0], gsems[0])
        for c in range(_NCH_D):
            b = c % 2
            g_h[c].wait()
            if c + 1 < _NCH_D:
                if c >= 1:
                    s_h[c - 1].wait()
                g_h[c + 1] = pltpu.async_copy(
                    x_hbm.at[tok_idx(c + 1)], bufs[1 - b], gsems[1 - b])
            s_h[c] = pltpu.async_copy(bufs[b], out_hbm.at[dest_v.at[c]],
                                      ssems[b])
        s_h[_NCH_D - 2].wait()
        s_h[_NCH_D - 1].wait()

    return _sc_dispatch


def _mm_body(meta_ref, xs_ref, w1_ref, w3_ref, w2_ref, y_ref):
    g = pl.program_id(0)

    @pl.when(g < meta_ref[G_MAX])
    def _():
        xb = xs_ref[...]
        a = lax.dot_general(xb, w1_ref[0], (((1,), (1,)), ((), ())),
                            preferred_element_type=jnp.float32)
        b = lax.dot_general(xb, w3_ref[0], (((1,), (1,)), ((), ())),
                            preferred_element_type=jnp.float32)
        h = a * jax.nn.sigmoid(a) * b
        y = lax.dot_general(h, w2_ref[0], (((1,), (1,)), ((), ())),
                            preferred_element_type=jnp.float32)
        y_ref[...] = y


def _grouped_mm(meta, xs, W1, W3, W2):
    def io_map(g, m):
        return (jnp.minimum(g, m[G_MAX] - 1), 0)

    grid_spec = pltpu.PrefetchScalarGridSpec(
        num_scalar_prefetch=1,
        grid=(G_MAX,),
        in_specs=[
            pl.BlockSpec((BT, D_MODEL), io_map),
            pl.BlockSpec((1, HIDDEN, D_MODEL), lambda g, m: (m[g], 0, 0)),
            pl.BlockSpec((1, HIDDEN, D_MODEL), lambda g, m: (m[g], 0, 0)),
            pl.BlockSpec((1, D_MODEL, HIDDEN), lambda g, m: (m[g], 0, 0)),
        ],
        out_specs=pl.BlockSpec((BT, D_MODEL), io_map),
    )
    return pl.pallas_call(
        _mm_body,
        grid_spec=grid_spec,
        out_shape=jax.ShapeDtypeStruct((P, D_MODEL), jnp.float32),
        compiler_params=pltpu.CompilerParams(
            dimension_semantics=("arbitrary",)),
    )(meta, xs, W1, W3, W2)


_TOK_W = N_TOK // NW
_CH_C = 16
_NCH_C = _TOK_W // _CH_C
_NV = D_MODEL // 16
_UNROLL = 8


@functools.cache
def _make_sc_combine():
    @functools.partial(
        pl.kernel,
        out_type=jax.ShapeDtypeStruct((N_TOK, D_MODEL), jnp.float32),
        mesh=_sc_mesh(),
        scratch_types=[
            pltpu.VMEM((_TOK_W,), jnp.int32),
            pltpu.VMEM((_TOK_W,), jnp.int32),
            pltpu.VMEM((_TOK_W,), jnp.float32),
            pltpu.VMEM((_TOK_W,), jnp.float32),
            pltpu.VMEM((_CH_C, D_MODEL), jnp.float32),
            pltpu.VMEM((_CH_C, D_MODEL), jnp.float32),
            pltpu.VMEM((_CH_C, D_MODEL), jnp.float32),
            pltpu.VMEM((_CH_C, D_MODEL), jnp.float32),
            pltpu.SemaphoreType.DMA,
            pltpu.SemaphoreType.DMA,
            pltpu.SemaphoreType.DMA,
            pltpu.SemaphoreType.DMA,
            pltpu.SemaphoreType.DMA,
            pltpu.SemaphoreType.DMA,
        ],
    )
    def _sc_combine(y_hbm, pos0_hbm, pos1_hbm, w0_hbm, w1_hbm, out_hbm,
                    idx0_v, idx1_v, w0_v, w1_v, a0, a1, b0, b1,
                    gs0a, gs1a, gs0b, gs1b, wsa, wsb):
        wid = lax.axis_index("s") * NC + lax.axis_index("c")
        base = wid * _TOK_W
        pltpu.sync_copy(pos0_hbm.at[pl.ds(base, _TOK_W)], idx0_v)
        pltpu.sync_copy(pos1_hbm.at[pl.ds(base, _TOK_W)], idx1_v)
        pltpu.sync_copy(w0_hbm.at[pl.ds(base, _TOK_W)], w0_v)
        pltpu.sync_copy(w1_hbm.at[pl.ds(base, _TOK_W)], w1_v)
        e0 = (a0, b0)
        e1 = (a1, b1)
        gs0 = (gs0a, gs0b)
        gs1 = (gs1a, gs1b)
        ws = (wsa, wsb)

        def start_gathers(c, b):
            sl = pl.ds(c * _CH_C, _CH_C)
            return (pltpu.async_copy(y_hbm.at[idx0_v.at[sl]], e0[b], gs0[b]),
                    pltpu.async_copy(y_hbm.at[idx1_v.at[sl]], e1[b], gs1[b]))

        g_h = {0: start_gathers(0, 0)}
        w_h = {}
        for c in range(_NCH_C):
            b = c % 2
            g_h[c][0].wait()
            g_h[c][1].wait()
            if c + 1 < _NCH_C:
                if c >= 1:
                    w_h[c - 1].wait()
                g_h[c + 1] = start_gathers(c + 1, 1 - b)
            wvec0 = w0_v[pl.ds(c * _CH_C, _CH_C)]
            wvec1 = w1_v[pl.ds(c * _CH_C, _CH_C)]
            for t in range(_CH_C):
                w0b = wvec0[t]
                w1b = wvec1[t]

                def body(j, _, t=t, w0b=w0b, w1b=w1b, b=b):
                    for u in range(_UNROLL):
                        vsl = pl.ds(j * (16 * _UNROLL) + u * 16, 16)
                        e0[b][t, vsl] = (e0[b][t, vsl] * w0b
                                         + e1[b][t, vsl] * w1b)
                    return 0
                lax.fori_loop(0, _NV // _UNROLL, body, 0)
            w_h[c] = pltpu.async_copy(
                e0[b], out_hbm.at[pl.ds(base + c * _CH_C, _CH_C)], ws[b])
        w_h[_NCH_C - 2].wait()
        w_h[_NCH_C - 1].wait()

    return _sc_combine


def kernel(x, Wr, W1, W3, W2):
    bsz, seq_len, d_model = x.shape
    x_flat = x.reshape(-1, d_model)

    dest32, top_w, binfo = _router(x_flat, Wr)
    bexcl = binfo[0]
    eob = jnp.sum((jnp.arange(G_MAX)[:, None] >= bexcl[None, :])
                  .astype(jnp.int32), axis=1) - 1
    meta = jnp.concatenate([eob, binfo[1, N_EXPERTS - 1:]]).astype(jnp.int32)

    dest3 = dest32.reshape(NW, _NCH_D, _CH_D)
    dflat = dest32.reshape(A)
    pos0 = dflat[:N_TOK]
    pos1 = dflat[N_TOK:]

    xs = _make_sc_dispatch()(x_flat, dest3)
    ys = _grouped_mm(meta, xs, W1, W3, W2)
    out = _make_sc_combine()(ys, pos0, pos1,
                             top_w[:, 0], top_w[:, 1])
    return out.reshape(bsz, seq_len, d_model)


# speedup vs baseline: 1.3932x; 1.0071x over previous
import functools

import jax
import jax.numpy as jnp
from jax import lax
from jax.experimental import pallas as pl
from jax.experimental.pallas import tpu as pltpu
from jax.experimental.pallas import tpu_sc as plsc

D_MODEL = 1024
HIDDEN = 2048
N_EXPERTS = 8
TOP_K = 2
N_TOK = 2048
A = N_TOK * TOP_K
BT = 256
G_MAX = A // BT + N_EXPERTS
P = G_MAX * BT

NC = 2
NS = 16
NW = NC * NS


def _sc_mesh():
    return plsc.VectorSubcoreMesh(core_axis_name="c", subcore_axis_name="s",
                                  num_cores=NC, num_subcores=NS)


_NCHUNK = A // 128


def _router_body(x_ref, wr_ref, dest_ref, w_ref, meta_ref):
    x = x_ref[...]
    wr = wr_ref[...]
    logits = lax.dot_general(x, wr, (((1,), (1,)), ((), ())),
                             preferred_element_type=jnp.float32)
    iota = lax.broadcasted_iota(jnp.int32, logits.shape, 1)
    m1 = jnp.max(logits, axis=1, keepdims=True)
    i1 = jnp.min(jnp.where(logits == m1, iota, N_EXPERTS), axis=1, keepdims=True)
    masked = jnp.where(iota == i1, -jnp.inf, logits)
    m2 = jnp.max(masked, axis=1, keepdims=True)
    i2 = jnp.min(jnp.where(masked == m2, iota, N_EXPERTS), axis=1, keepdims=True)
    e2 = jnp.exp(m2 - m1)
    denom = 1.0 + e2
    w_ref[...] = jnp.concatenate([1.0 / denom, e2 / denom], axis=1)

    e_col = jnp.concatenate([i1, i2], axis=0)
    e3 = e_col.reshape(_NCHUNK, 128, 1)
    iota_e = lax.broadcasted_iota(jnp.int32, (_NCHUNK, 128, N_EXPERTS), 2)
    oh = (e3 == iota_e).astype(jnp.float32)
    ii = lax.broadcasted_iota(jnp.int32, (128, 128), 0)
    jj = lax.broadcasted_iota(jnp.int32, (128, 128), 1)
    tril = (jj < ii).astype(jnp.bfloat16)
    tril_b = jnp.broadcast_to(tril[None], (_NCHUNK, 128, 128))
    excl = lax.dot_general(tril_b, oh.astype(jnp.bfloat16),
                           (((2,), (1,)), ((0,), (0,))),
                           preferred_element_type=jnp.float32)
    chunk_sums = excl[:, 127, :] + oh[:, 127, :]
    ii32 = lax.broadcasted_iota(jnp.int32, (_NCHUNK, _NCHUNK), 0)
    jj32 = lax.broadcasted_iota(jnp.int32, (_NCHUNK, _NCHUNK), 1)
    tril32 = (jj32 < ii32).astype(jnp.bfloat16)
    offs = lax.dot_general(tril32, chunk_sums.astype(jnp.bfloat16),
                           (((1,), (0,)), ((), ())),
                           preferred_element_type=jnp.float32)
    counts = jnp.sum(chunk_sums, axis=0, keepdims=True)
    nb = (counts.astype(jnp.int32) + (BT - 1)) >> 8
    nb_f = nb.astype(jnp.float32)
    ii8 = lax.broadcasted_iota(jnp.int32, (N_EXPERTS, N_EXPERTS), 0)
    jj8 = lax.broadcasted_iota(jnp.int32, (N_EXPERTS, N_EXPERTS), 1)
    sup8 = (ii8 < jj8).astype(jnp.float32)
    bexcl = lax.dot_general(nb_f, sup8, (((1,), (0,)), ((), ())),
                            preferred_element_type=jnp.float32)
    base = float(BT) * bexcl
    dest3 = excl + offs[:, None, :] + base[None, :, :]
    dest_ref[...] = jnp.sum(dest3 * oh, axis=2).astype(jnp.int32)

    total = jnp.sum(nb_f, axis=1, keepdims=True)
    gi = lax.broadcasted_iota(jnp.int32, (32, N_EXPERTS), 0)
    eob = jnp.sum((gi.astype(jnp.float32) >= bexcl).astype(jnp.float32),
                  axis=1, keepdims=True) - 1.0
    gi0 = lax.broadcasted_iota(jnp.int32, (32, 1), 0)
    meta_ref[...] = jnp.where(gi0 == G_MAX, total, eob).astype(jnp.int32)


def _router(x_flat, Wr):
    return pl.pallas_call(
        _router_body,
        out_shape=(
            jax.ShapeDtypeStruct((_NCHUNK, 128), jnp.int32),
            jax.ShapeDtypeStruct((N_TOK, TOP_K), jnp.float32),
            jax.ShapeDtypeStruct((32, 1), jnp.int32),
        ),
    )(x_flat, Wr)


_AS_W = A // NW
_CH_D = 16
_NCH_D = _AS_W // _CH_D


@functools.cache
def _make_sc_dispatch():
    @functools.partial(
        pl.kernel,
        out_type=jax.ShapeDtypeStruct((P, D_MODEL), jnp.float32),
        mesh=_sc_mesh(),
        scratch_types=[
            pltpu.VMEM((_NCH_D, _CH_D), jnp.int32),
            pltpu.VMEM((_CH_D, D_MODEL), jnp.float32),
            pltpu.VMEM((_CH_D, D_MODEL), jnp.float32),
            pltpu.SemaphoreType.DMA,
            pltpu.SemaphoreType.DMA,
            pltpu.SemaphoreType.DMA,
            pltpu.SemaphoreType.DMA,
        ],
    )
    def _sc_dispatch(x_hbm, dest_hbm, out_hbm, dest_v, bufa, bufb,
                     gsa, gsb, ssa, ssb):
        wid = lax.axis_index("s") * NC + lax.axis_index("c")
        a_base = wid * _AS_W
        pltpu.sync_copy(dest_hbm.at[wid], dest_v)
        bufs = (bufa, bufb)
        gsems = (gsa, gsb)
        ssems = (ssa, ssb)

        def tok_idx(c):
            a_vec = (a_base + c * _CH_D) + lax.iota(jnp.int32, 16)
            return lax.bitwise_and(a_vec, N_TOK - 1)

        g_h = {}
        s_h = {}
        g_h[0] = pltpu.async_copy(x_hbm.at[tok_idx(0)], bufs[0], gsems[0])
        for c in range(_NCH_D):
            b = c % 2
            g_h[c].wait()
            if c + 1 < _NCH_D:
                if c >= 1:
                    s_h[c - 1].wait()
                g_h[c + 1] = pltpu.async_copy(
                    x_hbm.at[tok_idx(c + 1)], bufs[1 - b], gsems[1 - b])
            s_h[c] = pltpu.async_copy(bufs[b], out_hbm.at[dest_v.at[c]],
                                      ssems[b])
        s_h[_NCH_D - 2].wait()
        s_h[_NCH_D - 1].wait()

    return _sc_dispatch


def _mm_body(meta_ref, xs_ref, w1_ref, w3_ref, w2_ref, y_ref):
    g = pl.program_id(0)

    @pl.when(g < meta_ref[G_MAX])
    def _():
        xb = xs_ref[...]
        a = lax.dot_general(xb, w1_ref[0], (((1,), (1,)), ((), ())),
                            preferred_element_type=jnp.float32)
        b = lax.dot_general(xb, w3_ref[0], (((1,), (1,)), ((), ())),
                            preferred_element_type=jnp.float32)
        h = a * jax.nn.sigmoid(a) * b
        y = lax.dot_general(h, w2_ref[0], (((1,), (1,)), ((), ())),
                            preferred_element_type=jnp.float32)
        y_ref[...] = y


def _grouped_mm(meta, xs, W1, W3, W2):
    def io_map(g, m):
        return (jnp.minimum(g, m[G_MAX] - 1), 0)

    grid_spec = pltpu.PrefetchScalarGridSpec(
        num_scalar_prefetch=1,
        grid=(G_MAX,),
        in_specs=[
            pl.BlockSpec((BT, D_MODEL), io_map),
            pl.BlockSpec((1, HIDDEN, D_MODEL), lambda g, m: (m[g], 0, 0)),
            pl.BlockSpec((1, HIDDEN, D_MODEL), lambda g, m: (m[g], 0, 0)),
            pl.BlockSpec((1, D_MODEL, HIDDEN), lambda g, m: (m[g], 0, 0)),
        ],
        out_specs=pl.BlockSpec((BT, D_MODEL), io_map),
    )
    return pl.pallas_call(
        _mm_body,
        grid_spec=grid_spec,
        out_shape=jax.ShapeDtypeStruct((P, D_MODEL), jnp.float32),
        compiler_params=pltpu.CompilerParams(
            dimension_semantics=("arbitrary",)),
    )(meta, xs, W1, W3, W2)


_TOK_W = N_TOK // NW
_CH_C = 16
_NCH_C = _TOK_W // _CH_C
_NV = D_MODEL // 16
_UNROLL = 8


@functools.cache
def _make_sc_combine():
    @functools.partial(
        pl.kernel,
        out_type=jax.ShapeDtypeStruct((N_TOK, D_MODEL), jnp.float32),
        mesh=_sc_mesh(),
        scratch_types=[
            pltpu.VMEM((_TOK_W,), jnp.int32),
            pltpu.VMEM((_TOK_W,), jnp.int32),
            pltpu.VMEM((_TOK_W,), jnp.float32),
            pltpu.VMEM((_TOK_W,), jnp.float32),
            pltpu.VMEM((_CH_C, D_MODEL), jnp.float32),
            pltpu.VMEM((_CH_C, D_MODEL), jnp.float32),
            pltpu.VMEM((_CH_C, D_MODEL), jnp.float32),
            pltpu.VMEM((_CH_C, D_MODEL), jnp.float32),
            pltpu.SemaphoreType.DMA,
            pltpu.SemaphoreType.DMA,
            pltpu.SemaphoreType.DMA,
            pltpu.SemaphoreType.DMA,
            pltpu.SemaphoreType.DMA,
            pltpu.SemaphoreType.DMA,
        ],
    )
    def _sc_combine(y_hbm, dest_hbm, w0_hbm, w1_hbm, out_hbm,
                    idx0_v, idx1_v, w0_v, w1_v, a0, a1, b0, b1,
                    gs0a, gs1a, gs0b, gs1b, wsa, wsb):
        wid = lax.axis_index("s") * NC + lax.axis_index("c")
        base = wid * _TOK_W
        r0 = lax.shift_right_logical(wid, 1)
        c0 = lax.bitwise_and(wid, 1) * _TOK_W
        pltpu.sync_copy(dest_hbm.at[r0, pl.ds(c0, _TOK_W)], idx0_v)
        pltpu.sync_copy(dest_hbm.at[(N_TOK // 128) + r0, pl.ds(c0, _TOK_W)],
                        idx1_v)
        pltpu.sync_copy(w0_hbm.at[pl.ds(base, _TOK_W)], w0_v)
        pltpu.sync_copy(w1_hbm.at[pl.ds(base, _TOK_W)], w1_v)
        e0 = (a0, b0)
        e1 = (a1, b1)
        gs0 = (gs0a, gs0b)
        gs1 = (gs1a, gs1b)
        ws = (wsa, wsb)

        def start_gathers(c, b):
            sl = pl.ds(c * _CH_C, _CH_C)
            return (pltpu.async_copy(y_hbm.at[idx0_v.at[sl]], e0[b], gs0[b]),
                    pltpu.async_copy(y_hbm.at[idx1_v.at[sl]], e1[b], gs1[b]))

        g_h = {0: start_gathers(0, 0)}
        w_h = {}
        for c in range(_NCH_C):
            b = c % 2
            g_h[c][0].wait()
            g_h[c][1].wait()
            if c + 1 < _NCH_C:
                if c >= 1:
                    w_h[c - 1].wait()
                g_h[c + 1] = start_gathers(c + 1, 1 - b)
            wvec0 = w0_v[pl.ds(c * _CH_C, _CH_C)]
            wvec1 = w1_v[pl.ds(c * _CH_C, _CH_C)]
            for t in range(_CH_C):
                w0b = wvec0[t]
                w1b = wvec1[t]

                def body(j, _, t=t, w0b=w0b, w1b=w1b, b=b):
                    for u in range(_UNROLL):
                        vsl = pl.ds(j * (16 * _UNROLL) + u * 16, 16)
                        e0[b][t, vsl] = (e0[b][t, vsl] * w0b
                                         + e1[b][t, vsl] * w1b)
                    return 0
                lax.fori_loop(0, _NV // _UNROLL, body, 0)
            w_h[c] = pltpu.async_copy(
                e0[b], out_hbm.at[pl.ds(base + c * _CH_C, _CH_C)], ws[b])
        w_h[_NCH_C - 2].wait()
        w_h[_NCH_C - 1].wait()

    return _sc_combine


def kernel(x, Wr, W1, W3, W2):
    bsz, seq_len, d_model = x.shape
    x_flat = x.reshape(-1, d_model)

    dest32, top_w, meta2d = _router(x_flat, Wr)
    meta = meta2d.reshape(32)
    dest3 = dest32.reshape(NW, _NCH_D, _CH_D)

    xs = _make_sc_dispatch()(x_flat, dest3)
    ys = _grouped_mm(meta, xs, W1, W3, W2)
    out = _make_sc_combine()(ys, dest32,
                             top_w[:, 0], top_w[:, 1])
    return out.reshape(bsz, seq_len, d_model)


# speedup vs baseline: 1.4104x; 1.0124x over previous
import functools

import jax
import jax.numpy as jnp
from jax import lax
from jax.experimental import pallas as pl
from jax.experimental.pallas import tpu as pltpu
from jax.experimental.pallas import tpu_sc as plsc

D_MODEL = 1024
HIDDEN = 2048
N_EXPERTS = 8
TOP_K = 2
N_TOK = 2048
A = N_TOK * TOP_K
BT = 256
G_MAX = A // BT + N_EXPERTS
P = G_MAX * BT

NC = 2
NS = 16
NW = NC * NS


def _sc_mesh():
    return plsc.VectorSubcoreMesh(core_axis_name="c", subcore_axis_name="s",
                                  num_cores=NC, num_subcores=NS)


_NCHUNK = A // 128


def _router_body(x_ref, wr_ref, dest_ref, w_ref, meta_ref):
    x = x_ref[...]
    wr = wr_ref[...]
    logits = lax.dot_general(x, wr, (((1,), (1,)), ((), ())),
                             preferred_element_type=jnp.float32)
    iota = lax.broadcasted_iota(jnp.int32, logits.shape, 1)
    m1 = jnp.max(logits, axis=1, keepdims=True)
    i1 = jnp.min(jnp.where(logits == m1, iota, N_EXPERTS), axis=1, keepdims=True)
    masked = jnp.where(iota == i1, -jnp.inf, logits)
    m2 = jnp.max(masked, axis=1, keepdims=True)
    i2 = jnp.min(jnp.where(masked == m2, iota, N_EXPERTS), axis=1, keepdims=True)
    e2 = jnp.exp(m2 - m1)
    denom = 1.0 + e2
    w_ref[...] = jnp.concatenate([1.0 / denom, e2 / denom], axis=1)

    e_col = jnp.concatenate([i1, i2], axis=0)
    e3 = e_col.reshape(_NCHUNK, 128, 1)
    iota_e = lax.broadcasted_iota(jnp.int32, (_NCHUNK, 128, N_EXPERTS), 2)
    oh = (e3 == iota_e).astype(jnp.float32)
    ii = lax.broadcasted_iota(jnp.int32, (128, 128), 0)
    jj = lax.broadcasted_iota(jnp.int32, (128, 128), 1)
    tril = (jj < ii).astype(jnp.bfloat16)
    tril_b = jnp.broadcast_to(tril[None], (_NCHUNK, 128, 128))
    excl = lax.dot_general(tril_b, oh.astype(jnp.bfloat16),
                           (((2,), (1,)), ((0,), (0,))),
                           preferred_element_type=jnp.float32)
    chunk_sums = excl[:, 127, :] + oh[:, 127, :]
    ii32 = lax.broadcasted_iota(jnp.int32, (_NCHUNK, _NCHUNK), 0)
    jj32 = lax.broadcasted_iota(jnp.int32, (_NCHUNK, _NCHUNK), 1)
    tril32 = (jj32 < ii32).astype(jnp.bfloat16)
    offs = lax.dot_general(tril32, chunk_sums.astype(jnp.bfloat16),
                           (((1,), (0,)), ((), ())),
                           preferred_element_type=jnp.float32)
    counts = jnp.sum(chunk_sums, axis=0, keepdims=True)
    nb = (counts.astype(jnp.int32) + (BT - 1)) >> 8
    nb_f = nb.astype(jnp.float32)
    ii8 = lax.broadcasted_iota(jnp.int32, (N_EXPERTS, N_EXPERTS), 0)
    jj8 = lax.broadcasted_iota(jnp.int32, (N_EXPERTS, N_EXPERTS), 1)
    sup8 = (ii8 < jj8).astype(jnp.float32)
    bexcl = lax.dot_general(nb_f, sup8, (((1,), (0,)), ((), ())),
                            preferred_element_type=jnp.float32)
    base = float(BT) * bexcl
    dest3 = excl + offs[:, None, :] + base[None, :, :]
    dest_ref[...] = jnp.sum(dest3 * oh, axis=2).astype(jnp.int32)

    total = jnp.sum(nb_f, axis=1, keepdims=True)
    gi = lax.broadcasted_iota(jnp.int32, (32, N_EXPERTS), 0)
    eob = jnp.sum((gi.astype(jnp.float32) >= bexcl).astype(jnp.float32),
                  axis=1, keepdims=True) - 1.0
    gi0 = lax.broadcasted_iota(jnp.int32, (32, 1), 0)
    meta_ref[...] = jnp.where(gi0 == G_MAX, total, eob).astype(jnp.int32)


def _router(x_flat, Wr):
    return pl.pallas_call(
        _router_body,
        out_shape=(
            jax.ShapeDtypeStruct((_NCHUNK, 128), jnp.int32),
            jax.ShapeDtypeStruct((N_TOK, TOP_K), jnp.float32),
            jax.ShapeDtypeStruct((32, 1), jnp.int32),
        ),
    )(x_flat, Wr)


_AS_W = A // NW
_CH_D = 16
_NCH_D = _AS_W // _CH_D
_NB_D = 3


@functools.cache
def _make_sc_dispatch():
    @functools.partial(
        pl.kernel,
        out_type=jax.ShapeDtypeStruct((P, D_MODEL), jnp.float32),
        mesh=_sc_mesh(),
        scratch_types=(
            [pltpu.VMEM((_NCH_D, _CH_D), jnp.int32)]
            + [pltpu.VMEM((_CH_D, D_MODEL), jnp.float32)] * _NB_D
            + [pltpu.SemaphoreType.DMA] * (2 * _NB_D)
        ),
    )
    def _sc_dispatch(x_hbm, dest_hbm, out_hbm, dest_v, *scr):
        bufs = scr[:_NB_D]
        gsems = scr[_NB_D:2 * _NB_D]
        ssems = scr[2 * _NB_D:]
        wid = lax.axis_index("s") * NC + lax.axis_index("c")
        a_base = wid * _AS_W
        pltpu.sync_copy(dest_hbm.at[wid], dest_v)

        def tok_idx(c):
            a_vec = (a_base + c * _CH_D) + lax.iota(jnp.int32, 16)
            return lax.bitwise_and(a_vec, N_TOK - 1)

        def gather(c):
            b = c % _NB_D
            return pltpu.async_copy(x_hbm.at[tok_idx(c)], bufs[b], gsems[b])

        g_h = {}
        s_h = {}
        for c in range(min(_NB_D, _NCH_D)):
            g_h[c] = gather(c)
        for c in range(_NCH_D):
            b = c % _NB_D
            if c >= 1 and c - 1 + _NB_D < _NCH_D:
                s_h[c - 1].wait()
                g_h[c - 1 + _NB_D] = gather(c - 1 + _NB_D)
            g_h[c].wait()
            s_h[c] = pltpu.async_copy(bufs[b], out_hbm.at[dest_v.at[c]],
                                      ssems[b])
        for c in range(max(0, _NCH_D - _NB_D), _NCH_D):
            s_h[c].wait()

    return _sc_dispatch


def _mm_body(meta_ref, xs_ref, w1_ref, w3_ref, w2_ref, y_ref):
    g = pl.program_id(0)

    @pl.when(g < meta_ref[G_MAX])
    def _():
        xb = xs_ref[...]
        a = lax.dot_general(xb, w1_ref[0], (((1,), (1,)), ((), ())),
                            preferred_element_type=jnp.float32)
        b = lax.dot_general(xb, w3_ref[0], (((1,), (1,)), ((), ())),
                            preferred_element_type=jnp.float32)
        h = a * jax.nn.sigmoid(a) * b
        y = lax.dot_general(h, w2_ref[0], (((1,), (1,)), ((), ())),
                            preferred_element_type=jnp.float32)
        y_ref[...] = y


def _grouped_mm(meta, xs, W1, W3, W2):
    def io_map(g, m):
        return (jnp.minimum(g, m[G_MAX] - 1), 0)

    grid_spec = pltpu.PrefetchScalarGridSpec(
        num_scalar_prefetch=1,
        grid=(G_MAX,),
        in_specs=[
            pl.BlockSpec((BT, D_MODEL), io_map),
            pl.BlockSpec((1, HIDDEN, D_MODEL), lambda g, m: (m[g], 0, 0)),
            pl.BlockSpec((1, HIDDEN, D_MODEL), lambda g, m: (m[g], 0, 0)),
            pl.BlockSpec((1, D_MODEL, HIDDEN), lambda g, m: (m[g], 0, 0)),
        ],
        out_specs=pl.BlockSpec((BT, D_MODEL), io_map),
    )
    return pl.pallas_call(
        _mm_body,
        grid_spec=grid_spec,
        out_shape=jax.ShapeDtypeStruct((P, D_MODEL), jnp.float32),
        compiler_params=pltpu.CompilerParams(
            dimension_semantics=("arbitrary",)),
    )(meta, xs, W1, W3, W2)


_TOK_W = N_TOK // NW
_CH_C = 16
_NCH_C = _TOK_W // _CH_C
_NB_C = 3
_NV = D_MODEL // 16
_UNROLL = 8


@functools.cache
def _make_sc_combine():
    @functools.partial(
        pl.kernel,
        out_type=jax.ShapeDtypeStruct((N_TOK, D_MODEL), jnp.float32),
        mesh=_sc_mesh(),
        scratch_types=(
            [pltpu.VMEM((_TOK_W,), jnp.int32)] * 2
            + [pltpu.VMEM((_TOK_W,), jnp.float32)] * 2
            + [pltpu.VMEM((_CH_C, D_MODEL), jnp.float32)] * (2 * _NB_C)
            + [pltpu.SemaphoreType.DMA] * (3 * _NB_C)
        ),
    )
    def _sc_combine(y_hbm, dest_hbm, w0_hbm, w1_hbm, out_hbm,
                    idx0_v, idx1_v, w0_v, w1_v, *scr):
        e0 = scr[:_NB_C]
        e1 = scr[_NB_C:2 * _NB_C]
        gs0 = scr[2 * _NB_C:3 * _NB_C]
        gs1 = scr[3 * _NB_C:4 * _NB_C]
        ws = scr[4 * _NB_C:5 * _NB_C]
        wid = lax.axis_index("s") * NC + lax.axis_index("c")
        base = wid * _TOK_W
        r0 = lax.shift_right_logical(wid, 1)
        c0 = lax.bitwise_and(wid, 1) * _TOK_W
        pltpu.sync_copy(dest_hbm.at[r0, pl.ds(c0, _TOK_W)], idx0_v)
        pltpu.sync_copy(dest_hbm.at[(N_TOK // 128) + r0, pl.ds(c0, _TOK_W)],
                        idx1_v)
        pltpu.sync_copy(w0_hbm.at[pl.ds(base, _TOK_W)], w0_v)
        pltpu.sync_copy(w1_hbm.at[pl.ds(base, _TOK_W)], w1_v)

        def start_gathers(c):
            b = c % _NB_C
            sl = pl.ds(c * _CH_C, _CH_C)
            return (pltpu.async_copy(y_hbm.at[idx0_v.at[sl]], e0[b], gs0[b]),
                    pltpu.async_copy(y_hbm.at[idx1_v.at[sl]], e1[b], gs1[b]))

        g_h = {}
        w_h = {}
        for c in range(min(_NB_C, _NCH_C)):
            g_h[c] = start_gathers(c)
        for c in range(_NCH_C):
            b = c % _NB_C
            if c >= 1 and c - 1 + _NB_C < _NCH_C:
                w_h[c - 1].wait()
                g_h[c - 1 + _NB_C] = start_gathers(c - 1 + _NB_C)
            g_h[c][0].wait()
            g_h[c][1].wait()
            wvec0 = w0_v[pl.ds(c * _CH_C, _CH_C)]
            wvec1 = w1_v[pl.ds(c * _CH_C, _CH_C)]
            for t in range(_CH_C):
                w0b = wvec0[t]
                w1b = wvec1[t]

                def body(j, _, t=t, w0b=w0b, w1b=w1b, b=b):
                    for u in range(_UNROLL):
                        vsl = pl.ds(j * (16 * _UNROLL) + u * 16, 16)
                        e0[b][t, vsl] = (e0[b][t, vsl] * w0b
                                         + e1[b][t, vsl] * w1b)
                    return 0
                lax.fori_loop(0, _NV // _UNROLL, body, 0)
            w_h[c] = pltpu.async_copy(
                e0[b], out_hbm.at[pl.ds(base + c * _CH_C, _CH_C)], ws[b])
        for c in range(max(0, _NCH_C - _NB_C), _NCH_C):
            w_h[c].wait()

    return _sc_combine


def kernel(x, Wr, W1, W3, W2):
    bsz, seq_len, d_model = x.shape
    x_flat = x.reshape(-1, d_model)

    dest32, top_w, meta2d = _router(x_flat, Wr)
    meta = meta2d.reshape(32)
    dest3 = dest32.reshape(NW, _NCH_D, _CH_D)

    xs = _make_sc_dispatch()(x_flat, dest3)
    ys = _grouped_mm(meta, xs, W1, W3, W2)
    out = _make_sc_combine()(ys, dest32,
                             top_w[:, 0], top_w[:, 1])
    return out.reshape(bsz, seq_len, d_model)
